# baseline (speedup 1.0000x reference)
import jax
import jax.numpy as jnp
from jax.experimental import pallas as pl

_TOPK = 8
_ROWS = 32
_NSPLIT = 4


def _fused_body(e0_ref, e1_ref, e2_ref, e3_ref, mask_ref, w_ref, b_ref,
                pooled_ref, attn_ref):
    e_refs = (e0_ref, e1_ref, e2_ref, e3_ref)
    KQ, D = e0_ref.shape[2], e0_ref.shape[3]
    K = KQ * _NSPLIT
    C = K // _ROWS
    RQ = KQ // C
    w = w_ref[...]
    parts = []
    for er in e_refs:
        sq = jax.lax.dot_general(
            er[0, 0], w,
            dimension_numbers=(((1,), (0,)), ((), ())),
            precision=jax.lax.Precision.DEFAULT,
            preferred_element_type=jnp.float32,
        )
        parts.append(sq.reshape(RQ, C))
    s = jnp.concatenate(parts, axis=0)
    s = s + b_ref[0, 0]
    m = mask_ref[0]
    s = jnp.where(m == 0.0, -jnp.inf, s)

    row_i = jax.lax.broadcasted_iota(jnp.int32, (_ROWS, C), 0)
    col_i = jax.lax.broadcasted_iota(jnp.int32, (_ROWS, C), 1)
    gidx = row_i * C + col_i
    s_work = jnp.maximum(s, jnp.float32(-3.0e38))
    attn = jnp.zeros((_ROWS, C), dtype=jnp.float32)
    pooled = jnp.zeros((1, D), dtype=jnp.float32)
    inv_k = jnp.float32(1.0 / _TOPK)
    for _ in range(_TOPK):
        v = jnp.max(s_work)
        cand = jnp.where(s_work == v, gidx, K)
        idx = jnp.min(cand)
        sel = gidx == idx
        attn = attn + jnp.where(sel, inv_k, 0.0)
        s_work = jnp.where(sel, -jnp.inf, s_work)
        q = idx // KQ
        local = idx - q * KQ
        row = jnp.zeros((1, D), dtype=jnp.float32)
        for qi, er in enumerate(e_refs):
            rq = er[0, 0, pl.ds(local, 1), :]
            row = row + jnp.where(q == qi, 1.0, 0.0) * rq
        pooled = pooled + row * inv_k
    pooled_ref[0] = pooled
    attn_ref[0] = attn


def kernel(embeddings, mask, W, b):
    B, K, D = embeddings.shape
    C = K // _ROWS
    KQ = K // _NSPLIT
    b2 = b.reshape(1, 1)
    w_t = W.reshape(D, 1)
    e4 = embeddings.reshape(B, _NSPLIT, KQ, D)
    mask4 = mask.reshape(B, _ROWS, C)
    e_specs = [
        pl.BlockSpec((1, 1, KQ, D), lambda i, qq=q: (i, qq, 0, 0))
        for q in range(_NSPLIT)
    ]
    pooled, attn = pl.pallas_call(
        _fused_body,
        grid=(B,),
        in_specs=e_specs + [
            pl.BlockSpec((1, _ROWS, C), lambda i: (i, 0, 0)),
            pl.BlockSpec((D, 1), lambda i: (0, 0)),
            pl.BlockSpec((1, 1), lambda i: (0, 0)),
        ],
        out_specs=[
            pl.BlockSpec((1, 1, D), lambda i: (i, 0, 0)),
            pl.BlockSpec((1, _ROWS, C), lambda i: (i, 0, 0)),
        ],
        out_shape=[
            jax.ShapeDtypeStruct((B, 1, D), jnp.float32),
            jax.ShapeDtypeStruct((B, _ROWS, C), jnp.float32),
        ],
    )(e4, e4, e4, e4, mask4, w_t, b2)
    return (pooled.reshape(B, D), attn.reshape(B, K))
